# Initial kernel scaffold; baseline (speedup 1.0000x reference)
import jax
import jax.numpy as jnp
from jax.experimental import pallas as pl


def kernel(x, embedding_table):
    raise NotImplementedError("write your pallas kernel here")


# speedup vs baseline: 4.0945x; 4.0945x over previous
import functools

import jax
import jax.numpy as jnp
import numpy as np
from jax import lax
from jax.experimental import pallas as pl
from jax.experimental.pallas import tpu as pltpu
from jax.experimental.pallas import tpu_sc as plsc

BATCH = 1024
MAX_LEN = 200
D_MODEL = 128
LANES = 16

NUM_TILES = 32
ROWS_TOTAL = BATCH * MAX_LEN
ROWS_PER_TILE = ROWS_TOTAL // NUM_TILES
CHUNK = MAX_LEN
NCHUNKS = ROWS_PER_TILE // CHUNK
GSPLIT = 128


def _positional_encoding_np():
    even_i = np.arange(0, D_MODEL, 2, dtype=np.float64)
    denominator = np.power(10000.0, 2.0 * even_i / D_MODEL)
    position = np.arange(MAX_LEN, dtype=np.float64).reshape(MAX_LEN, 1)
    even_pe = np.sin(position / denominator)
    odd_pe = np.cos(position / denominator)
    stacked = np.stack([even_pe, odd_pe], axis=2)
    return stacked.reshape(MAX_LEN, D_MODEL).astype(np.float32)


_PE = _positional_encoding_np()


@functools.partial(
    pl.kernel,
    out_type=jax.ShapeDtypeStruct((ROWS_TOTAL, D_MODEL), jnp.float32),
    mesh=plsc.VectorSubcoreMesh(core_axis_name="c", subcore_axis_name="s"),
    scratch_types=[
        pltpu.VMEM((ROWS_PER_TILE,), jnp.int32),
        pltpu.VMEM((MAX_LEN, D_MODEL), jnp.float32),
        pltpu.VMEM((CHUNK, D_MODEL), jnp.float32),
        pltpu.SemaphoreType.DMA,
        pltpu.SemaphoreType.DMA,
    ],
)
def _emb_kernel(x_hbm, table_hbm, pe_hbm, out_hbm, idx_v, pe_v, buf, g0, g1):
    wid = lax.axis_index("s") * 2 + lax.axis_index("c")
    base = wid * ROWS_PER_TILE

    pltpu.sync_copy(x_hbm.at[pl.ds(base, ROWS_PER_TILE)], idx_v)
    pltpu.sync_copy(pe_hbm, pe_v)

    def chunk_body(c, carry):
        roff = c * CHUNK
        cp1 = pltpu.make_async_copy(
            table_hbm.at[idx_v.at[pl.ds(roff, GSPLIT)]],
            buf.at[pl.ds(0, GSPLIT)],
            g0,
        )
        cp2 = pltpu.make_async_copy(
            table_hbm.at[idx_v.at[pl.ds(roff + GSPLIT, CHUNK - GSPLIT)]],
            buf.at[pl.ds(GSPLIT, CHUNK - GSPLIT)],
            g1,
        )
        cp1.start()
        cp2.start()
        cp1.wait()
        cp2.wait()

        def row_body(r, rcarry):
            for j in range(D_MODEL // LANES):
                pe_sl = pe_v[r, pl.ds(j * LANES, LANES)]
                plsc.addupdate(buf.at[r, pl.ds(j * LANES, LANES)], pe_sl)
            return rcarry

        lax.fori_loop(0, CHUNK, row_body, 0)

        pltpu.sync_copy(buf, out_hbm.at[pl.ds(base + roff, CHUNK)])
        return carry

    lax.fori_loop(0, NCHUNKS, chunk_body, 0)


def kernel(x, embedding_table):
    xf = x.reshape(ROWS_TOTAL)
    pe = jnp.asarray(_PE)
    out = _emb_kernel(xf, embedding_table, pe)
    return out.reshape(BATCH, MAX_LEN, D_MODEL)


# speedup vs baseline: 5.6146x; 1.3713x over previous
import functools

import jax
import jax.numpy as jnp
import numpy as np
from jax import lax
from jax.experimental import pallas as pl
from jax.experimental.pallas import tpu as pltpu
from jax.experimental.pallas import tpu_sc as plsc

BATCH = 1024
MAX_LEN = 200
D_MODEL = 128
LANES = 16

NUM_TILES = 32
ROWS_TOTAL = BATCH * MAX_LEN
ROWS_PER_TILE = ROWS_TOTAL // NUM_TILES
CHUNK = MAX_LEN
NCHUNKS = ROWS_PER_TILE // CHUNK
GSPLIT = 128


def _positional_encoding_np():
    even_i = np.arange(0, D_MODEL, 2, dtype=np.float64)
    denominator = np.power(10000.0, 2.0 * even_i / D_MODEL)
    position = np.arange(MAX_LEN, dtype=np.float64).reshape(MAX_LEN, 1)
    even_pe = np.sin(position / denominator)
    odd_pe = np.cos(position / denominator)
    stacked = np.stack([even_pe, odd_pe], axis=2)
    return stacked.reshape(MAX_LEN, D_MODEL).astype(np.float32)


_PE = _positional_encoding_np()


@functools.partial(
    pl.kernel,
    out_type=jax.ShapeDtypeStruct((ROWS_TOTAL, D_MODEL), jnp.float32),
    mesh=plsc.VectorSubcoreMesh(core_axis_name="c", subcore_axis_name="s"),
    scratch_types=[
        pltpu.VMEM((ROWS_PER_TILE,), jnp.int32),
        pltpu.VMEM((MAX_LEN, D_MODEL), jnp.float32),
        pltpu.VMEM((CHUNK, D_MODEL), jnp.float32),
        pltpu.VMEM((CHUNK, D_MODEL), jnp.float32),
        pltpu.SemaphoreType.DMA,
        pltpu.SemaphoreType.DMA,
        pltpu.SemaphoreType.DMA,
        pltpu.SemaphoreType.DMA,
    ],
)
def _emb_kernel(x_hbm, table_hbm, pe_hbm, out_hbm, idx_v, pe_v, buf0, buf1,
                g0, g1, s0, s1):
    wid = lax.axis_index("s") * 2 + lax.axis_index("c")
    base = wid * ROWS_PER_TILE

    pltpu.sync_copy(x_hbm.at[pl.ds(base, ROWS_PER_TILE)], idx_v)
    pltpu.sync_copy(pe_hbm, pe_v)

    def gather(c, buf, sem):
        roff = c * CHUNK
        pltpu.make_async_copy(
            table_hbm.at[idx_v.at[pl.ds(roff, GSPLIT)]],
            buf.at[pl.ds(0, GSPLIT)], sem).start()
        pltpu.make_async_copy(
            table_hbm.at[idx_v.at[pl.ds(roff + GSPLIT, CHUNK - GSPLIT)]],
            buf.at[pl.ds(GSPLIT, CHUNK - GSPLIT)], sem).start()

    def gather_wait(c, buf, sem):
        roff = c * CHUNK
        pltpu.make_async_copy(
            table_hbm.at[idx_v.at[pl.ds(roff, GSPLIT)]],
            buf.at[pl.ds(0, GSPLIT)], sem).wait()
        pltpu.make_async_copy(
            table_hbm.at[idx_v.at[pl.ds(roff + GSPLIT, CHUNK - GSPLIT)]],
            buf.at[pl.ds(GSPLIT, CHUNK - GSPLIT)], sem).wait()

    def add_pe(buf):
        def row_body(r, rcarry):
            for j in range(D_MODEL // LANES):
                pe_sl = pe_v[r, pl.ds(j * LANES, LANES)]
                plsc.addupdate(buf.at[r, pl.ds(j * LANES, LANES)], pe_sl)
            return rcarry
        lax.fori_loop(0, CHUNK, row_body, 0)

    def store(c, buf, sem):
        pltpu.make_async_copy(
            buf, out_hbm.at[pl.ds(base + c * CHUNK, CHUNK)], sem).start()

    def store_wait(c, buf, sem):
        pltpu.make_async_copy(
            buf, out_hbm.at[pl.ds(base + c * CHUNK, CHUNK)], sem).wait()

    gather(0, buf0, g0)

    def pair_body(i, carry):
        c0 = 2 * i
        gather_wait(c0, buf0, g0)

        @pl.when(i > 0)
        def _():
            store_wait(c0 - 1, buf1, s1)

        gather(c0 + 1, buf1, g1)
        add_pe(buf0)
        store(c0, buf0, s0)

        gather_wait(c0 + 1, buf1, g1)
        store_wait(c0, buf0, s0)
        nxt = jnp.minimum(c0 + 2, NCHUNKS - 1)
        gather(nxt, buf0, g0)
        add_pe(buf1)
        store(c0 + 1, buf1, s1)
        return carry

    lax.fori_loop(0, NCHUNKS // 2, pair_body, 0)
    gather_wait(NCHUNKS - 1, buf0, g0)
    store_wait(NCHUNKS - 1, buf1, s1)


def kernel(x, embedding_table):
    xf = x.reshape(ROWS_TOTAL)
    pe = jnp.asarray(_PE)
    out = _emb_kernel(xf, embedding_table, pe)
    return out.reshape(BATCH, MAX_LEN, D_MODEL)


# speedup vs baseline: 6.2655x; 1.1159x over previous
import functools

import jax
import jax.numpy as jnp
import numpy as np
from jax import lax
from jax.experimental import pallas as pl
from jax.experimental.pallas import tpu as pltpu
from jax.experimental.pallas import tpu_sc as plsc

BATCH = 1024
MAX_LEN = 200
D_MODEL = 128
VOCAB = 1000
LANES = 16

NUM_TILES = 32
ROWS_TOTAL = BATCH * MAX_LEN
ROWS_PER_TILE = ROWS_TOTAL // NUM_TILES
CHUNK = MAX_LEN
NCHUNKS = ROWS_PER_TILE // CHUNK
GSPLIT = 128


def _positional_encoding_np():
    even_i = np.arange(0, D_MODEL, 2, dtype=np.float64)
    denominator = np.power(10000.0, 2.0 * even_i / D_MODEL)
    position = np.arange(MAX_LEN, dtype=np.float64).reshape(MAX_LEN, 1)
    even_pe = np.sin(position / denominator)
    odd_pe = np.cos(position / denominator)
    stacked = np.stack([even_pe, odd_pe], axis=2)
    return stacked.reshape(MAX_LEN, D_MODEL).astype(np.float32)


_PE = _positional_encoding_np()


@functools.partial(
    pl.kernel,
    out_type=jax.ShapeDtypeStruct((ROWS_TOTAL, D_MODEL), jnp.float32),
    mesh=plsc.VectorSubcoreMesh(core_axis_name="c", subcore_axis_name="s"),
    scratch_types=[
        pltpu.VMEM((ROWS_PER_TILE,), jnp.int32),
        pltpu.VMEM((MAX_LEN, D_MODEL), jnp.float32),
        pltpu.VMEM((CHUNK, D_MODEL), jnp.float32),
        pltpu.VMEM((CHUNK, D_MODEL), jnp.float32),
        pltpu.VMEM_SHARED((VOCAB, D_MODEL), jnp.float32),
        pltpu.SemaphoreType.DMA,
        pltpu.SemaphoreType.DMA,
        pltpu.SemaphoreType.DMA,
        pltpu.SemaphoreType.DMA,
    ],
)
def _emb_kernel(x_hbm, table_hbm, pe_hbm, out_hbm, idx_v, pe_v, buf0, buf1,
                table_sp, g0, g1, s0, s1):
    sid = lax.axis_index("s")
    wid = sid * 2 + lax.axis_index("c")
    base = wid * ROWS_PER_TILE

    @pl.when(sid == 0)
    def _():
        pltpu.sync_copy(table_hbm, table_sp)

    pltpu.sync_copy(x_hbm.at[pl.ds(base, ROWS_PER_TILE)], idx_v)
    pltpu.sync_copy(pe_hbm, pe_v)
    plsc.subcore_barrier()

    def gather(c, buf, sem):
        roff = c * CHUNK
        pltpu.make_async_copy(
            table_sp.at[idx_v.at[pl.ds(roff, GSPLIT)]],
            buf.at[pl.ds(0, GSPLIT)], sem).start()
        pltpu.make_async_copy(
            table_sp.at[idx_v.at[pl.ds(roff + GSPLIT, CHUNK - GSPLIT)]],
            buf.at[pl.ds(GSPLIT, CHUNK - GSPLIT)], sem).start()

    def gather_wait(c, buf, sem):
        roff = c * CHUNK
        pltpu.make_async_copy(
            table_sp.at[idx_v.at[pl.ds(roff, GSPLIT)]],
            buf.at[pl.ds(0, GSPLIT)], sem).wait()
        pltpu.make_async_copy(
            table_sp.at[idx_v.at[pl.ds(roff + GSPLIT, CHUNK - GSPLIT)]],
            buf.at[pl.ds(GSPLIT, CHUNK - GSPLIT)], sem).wait()

    def add_pe(buf):
        def row_body(r, rcarry):
            for j in range(D_MODEL // LANES):
                pe_sl = pe_v[r, pl.ds(j * LANES, LANES)]
                plsc.addupdate(buf.at[r, pl.ds(j * LANES, LANES)], pe_sl)
            return rcarry
        lax.fori_loop(0, CHUNK, row_body, 0)

    def store(c, buf, sem):
        pltpu.make_async_copy(
            buf, out_hbm.at[pl.ds(base + c * CHUNK, CHUNK)], sem).start()

    def store_wait(c, buf, sem):
        pltpu.make_async_copy(
            buf, out_hbm.at[pl.ds(base + c * CHUNK, CHUNK)], sem).wait()

    gather(0, buf0, g0)

    def pair_body(i, carry):
        c0 = 2 * i
        gather_wait(c0, buf0, g0)

        @pl.when(i > 0)
        def _():
            store_wait(c0 - 1, buf1, s1)

        gather(c0 + 1, buf1, g1)
        add_pe(buf0)
        store(c0, buf0, s0)

        gather_wait(c0 + 1, buf1, g1)
        store_wait(c0, buf0, s0)
        nxt = jnp.minimum(c0 + 2, NCHUNKS - 1)
        gather(nxt, buf0, g0)
        add_pe(buf1)
        store(c0 + 1, buf1, s1)
        return carry

    lax.fori_loop(0, NCHUNKS // 2, pair_body, 0)
    gather_wait(NCHUNKS - 1, buf0, g0)
    store_wait(NCHUNKS - 1, buf1, s1)


def kernel(x, embedding_table):
    xf = x.reshape(ROWS_TOTAL)
    pe = jnp.asarray(_PE)
    out = _emb_kernel(xf, embedding_table, pe)
    return out.reshape(BATCH, MAX_LEN, D_MODEL)


# speedup vs baseline: 6.2850x; 1.0031x over previous
import functools

import jax
import jax.numpy as jnp
import numpy as np
from jax import lax
from jax.experimental import pallas as pl
from jax.experimental.pallas import tpu as pltpu
from jax.experimental.pallas import tpu_sc as plsc

BATCH = 1024
MAX_LEN = 200
D_MODEL = 128
VOCAB = 1000
LANES = 16

NUM_TILES = 32
ROWS_TOTAL = BATCH * MAX_LEN
ROWS_PER_TILE = ROWS_TOTAL // NUM_TILES
CHUNK = MAX_LEN
NCHUNKS = ROWS_PER_TILE // CHUNK
GSPLIT = 128


def _positional_encoding_np():
    even_i = np.arange(0, D_MODEL, 2, dtype=np.float64)
    denominator = np.power(10000.0, 2.0 * even_i / D_MODEL)
    position = np.arange(MAX_LEN, dtype=np.float64).reshape(MAX_LEN, 1)
    even_pe = np.sin(position / denominator)
    odd_pe = np.cos(position / denominator)
    stacked = np.stack([even_pe, odd_pe], axis=2)
    return stacked.reshape(MAX_LEN, D_MODEL).astype(np.float32)


_PE = _positional_encoding_np()


@functools.partial(
    pl.kernel,
    out_type=jax.ShapeDtypeStruct((ROWS_TOTAL, D_MODEL), jnp.float32),
    mesh=plsc.VectorSubcoreMesh(core_axis_name="c", subcore_axis_name="s"),
    scratch_types=[
        pltpu.VMEM((ROWS_PER_TILE,), jnp.int32),
        pltpu.VMEM((MAX_LEN, D_MODEL), jnp.float32),
        pltpu.VMEM((CHUNK, D_MODEL), jnp.float32),
        pltpu.VMEM((CHUNK, D_MODEL), jnp.float32),
        pltpu.VMEM_SHARED((VOCAB, D_MODEL), jnp.float32),
        pltpu.SemaphoreType.DMA,
        pltpu.SemaphoreType.DMA,
        pltpu.SemaphoreType.DMA,
        pltpu.SemaphoreType.DMA,
    ],
)
def _emb_kernel(x_hbm, table_hbm, pe_hbm, out_hbm, idx_v, pe_v, buf0, buf1,
                table_sp, g0, g1, s0, s1):
    sid = lax.axis_index("s")
    wid = sid * 2 + lax.axis_index("c")
    base = wid * ROWS_PER_TILE

    @pl.when(sid == 0)
    def _():
        pltpu.sync_copy(table_hbm, table_sp)

    pltpu.sync_copy(x_hbm.at[pl.ds(base, ROWS_PER_TILE)], idx_v)
    pltpu.sync_copy(pe_hbm, pe_v)
    plsc.subcore_barrier()

    def gather(c, buf, sem):
        roff = c * CHUNK
        pltpu.make_async_copy(
            table_sp.at[idx_v.at[pl.ds(roff, GSPLIT)]],
            buf.at[pl.ds(0, GSPLIT)], sem).start()
        pltpu.make_async_copy(
            table_sp.at[idx_v.at[pl.ds(roff + GSPLIT, CHUNK - GSPLIT)]],
            buf.at[pl.ds(GSPLIT, CHUNK - GSPLIT)], sem).start()

    def gather_wait(c, buf, sem):
        roff = c * CHUNK
        pltpu.make_async_copy(
            table_sp.at[idx_v.at[pl.ds(roff, GSPLIT)]],
            buf.at[pl.ds(0, GSPLIT)], sem).wait()
        pltpu.make_async_copy(
            table_sp.at[idx_v.at[pl.ds(roff + GSPLIT, CHUNK - GSPLIT)]],
            buf.at[pl.ds(GSPLIT, CHUNK - GSPLIT)], sem).wait()

    def add_pe(buf):
        @plsc.parallel_loop(0, CHUNK, step=1, unroll=4)
        def _(r):
            for j in range(D_MODEL // LANES):
                pe_sl = pe_v[r, pl.ds(j * LANES, LANES)]
                plsc.addupdate(buf.at[r, pl.ds(j * LANES, LANES)], pe_sl)

    def store(c, buf, sem):
        pltpu.make_async_copy(
            buf, out_hbm.at[pl.ds(base + c * CHUNK, CHUNK)], sem).start()

    def store_wait(c, buf, sem):
        pltpu.make_async_copy(
            buf, out_hbm.at[pl.ds(base + c * CHUNK, CHUNK)], sem).wait()

    gather(0, buf0, g0)

    def pair_body(i, carry):
        c0 = 2 * i
        gather_wait(c0, buf0, g0)

        @pl.when(i > 0)
        def _():
            store_wait(c0 - 1, buf1, s1)

        gather(c0 + 1, buf1, g1)
        add_pe(buf0)
        store(c0, buf0, s0)

        gather_wait(c0 + 1, buf1, g1)
        store_wait(c0, buf0, s0)
        nxt = jnp.minimum(c0 + 2, NCHUNKS - 1)
        gather(nxt, buf0, g0)
        add_pe(buf1)
        store(c0 + 1, buf1, s1)
        return carry

    lax.fori_loop(0, NCHUNKS // 2, pair_body, 0)
    gather_wait(NCHUNKS - 1, buf0, g0)
    store_wait(NCHUNKS - 1, buf1, s1)


def kernel(x, embedding_table):
    xf = x.reshape(ROWS_TOTAL)
    pe = jnp.asarray(_PE)
    out = _emb_kernel(xf, embedding_table, pe)
    return out.reshape(BATCH, MAX_LEN, D_MODEL)
